# baseline (speedup 1.0000x reference)
import functools

import jax
import jax.numpy as jnp
from jax import lax
from jax.experimental import pallas as pl
from jax.experimental.pallas import tpu as pltpu
from jax.experimental.pallas import tpu_sc as plsc

N = 10000
E = 320000
D = 64
D_IN = 128
D_EDGE_IN = 16

NC = 2
NS = 16
NW = NC * NS
EPW = E // NW
C = 80
NCHUNK = EPW // C
RPT = 632
N_PAD = NS * RPT
LANES = 16
VPR = D // LANES


def _pre_nodes_body(x_ref, we_ref, be_ref, wma_ref, out_ref, h_ref):
    o = jnp.dot(x_ref[...], we_ref[...], preferred_element_type=jnp.float32)
    o = jnp.maximum(o + be_ref[...], 0.0)
    out_ref[...] = o
    h_ref[...] = jnp.dot(o, wma_ref[...], preferred_element_type=jnp.float32)


def _pre_edges_body(lo_ref, hi_ref, wel_ref, bel_ref, wmb_ref, bm_ref,
                    ec_ref):
    def ec_half(ea):
        e = jnp.dot(ea, wel_ref[...], preferred_element_type=jnp.float32)
        e = jnp.maximum(e + bel_ref[...], 0.0)
        return jnp.dot(e, wmb_ref[...],
                       preferred_element_type=jnp.float32) + bm_ref[...]
    ec_ref[...] = jnp.concatenate(
        [ec_half(lo_ref[...]), ec_half(hi_ref[...])], axis=1)


def _update_body(out_ref, mp_ref, wut_ref, wub_ref, bu_ref, wma_ref,
                 outn_ref, hn_ref):
    m = mp_ref[0, :N] + mp_ref[1, :N]
    o = (jnp.dot(out_ref[...], wut_ref[...], preferred_element_type=jnp.float32)
         + jnp.dot(m, wub_ref[...], preferred_element_type=jnp.float32)
         + bu_ref[...])
    outn_ref[...] = o
    hn_ref[...] = jnp.dot(o, wma_ref[...], preferred_element_type=jnp.float32)


NBUF = 5


ZROWS = RPT - (RPT // C) * C


def _sc_step_body(h_hbm, ec_hbm, src_hbm, dst_hbm, out_hbm,
                  srcv, dstv, rows, macc, sem_g, sem_e, sem_s,
                  sem_i, sem_z):
    cid = lax.axis_index("c")
    sid = lax.axis_index("s")
    wid = sid * NC + cid

    cp_src = pltpu.async_copy(src_hbm.at[wid], srcv, sem_i)
    cp_dst = pltpu.async_copy(dst_hbm.at[wid], dstv, sem_i)

    def _zero(i, carry):
        rows[3, i // VPR, pl.ds((i % VPR) * LANES, LANES)] = jnp.zeros(
            (LANES,), jnp.float32)
        return carry
    lax.fori_loop(0, C * VPR, _zero, 0)

    def _zdma(z):
        if z < RPT // C:
            return pltpu.make_async_copy(
                rows.at[3], macc.at[pl.ds(sid * RPT + z * C, C)], sem_z)
        return pltpu.make_async_copy(
            rows.at[3, pl.ds(0, ZROWS)],
            macc.at[pl.ds(sid * RPT + (RPT // C) * C, ZROWS)], sem_z)
    for z in range(RPT // C + 1):
        _zdma(z).start()
    cp_src.wait()
    cp_dst.wait()

    def _drain_rows(sem, x):
        pltpu.make_async_copy(h_hbm.at[pl.ds(0, C)], rows.at[x], sem).wait()

    def _issue_ec(j, x):
        pltpu.async_copy(ec_hbm.at[wid, j], rows.at[x], sem_e.at[x])

    def _issue_gather(j, x):
        pltpu.async_copy(h_hbm.at[srcv.at[j]], rows.at[x], sem_g.at[x],
                         add=True)

    _issue_ec(0, 0)
    _issue_ec(1, 1)
    _drain_rows(sem_e.at[0], 0)
    _issue_gather(0, 0)
    for z in range(RPT // C + 1):
        _zdma(z).wait()
    plsc.subcore_barrier()

    def _outer(i, carry):
        j0 = i * NBUF
        for b in range(NBUF):
            j = j0 + b
            x2 = (b + 2) % NBUF
            x1 = (b + 1) % NBUF

            @pl.when(j + 2 < NCHUNK)
            def _():
                @pl.when(j >= 3)
                def _():
                    _drain_rows(sem_s.at[x2], x2)
                _issue_ec(j + 2, x2)

            @pl.when(j + 1 < NCHUNK)
            def _():
                _drain_rows(sem_e.at[x1], x1)
                _issue_gather(j + 1, x1)

            _drain_rows(sem_g.at[b], b)

            def _row(r, c2):
                for col in (0, 16, 32, 48):
                    rows[b, r, pl.ds(col, LANES)] = jnp.maximum(
                        rows[b, r, pl.ds(col, LANES)], 0.0)
                return c2
            lax.fori_loop(0, C, _row, 0, unroll=8)

            pltpu.async_copy(rows.at[b], macc.at[dstv.at[j]], sem_s.at[b],
                             add=True)
        return carry
    lax.fori_loop(0, NCHUNK // NBUF, _outer, 0)

    for b in range(NBUF):
        _drain_rows(sem_s.at[b], b)
    plsc.subcore_barrier()
    pltpu.sync_copy(macc.at[pl.ds(sid * RPT, RPT)],
                    out_hbm.at[cid, pl.ds(sid * RPT, RPT)])


_sc_step = functools.partial(
    pl.kernel,
    out_type=jax.ShapeDtypeStruct((NC, N_PAD, D), jnp.float32),
    mesh=plsc.VectorSubcoreMesh(core_axis_name="c", subcore_axis_name="s"),
    compiler_params=pltpu.CompilerParams(use_tc_tiling_on_sc=False),
    scratch_types=[
        pltpu.VMEM((NCHUNK, C), jnp.int32),
        pltpu.VMEM((NCHUNK, C), jnp.int32),
        pltpu.VMEM((NBUF, C, D), jnp.float32),
        pltpu.VMEM_SHARED((N_PAD, D), jnp.float32),
        pltpu.SemaphoreType.DMA((NBUF,)),
        pltpu.SemaphoreType.DMA((NBUF,)),
        pltpu.SemaphoreType.DMA((NBUF,)),
        pltpu.SemaphoreType.DMA,
        pltpu.SemaphoreType.DMA,
    ],
)(_sc_step_body)


def kernel(x, edge_index, edge_attr, We, be, Wel, bel, Wm, bm, Wu, bu):
    def perm(ix):
        ix = ix.astype(jnp.int32)
        ix = jnp.stack([ix[:E // 2], ix[E // 2:]], axis=1)
        return ix.reshape(NW, NCHUNK, C)
    src = perm(edge_index[0])
    dst = perm(edge_index[1])
    Wma, Wmb = Wm[:D], Wm[D:]
    Wut, Wub = Wu[:D], Wu[D:]
    be2 = be.reshape(1, D)
    bel2 = bel.reshape(1, D)
    bm2 = bm.reshape(1, D)
    bu2 = bu.reshape(1, D)

    NB = 1000
    out, h = pl.pallas_call(
        _pre_nodes_body,
        grid=(N // NB,),
        in_specs=[
            pl.BlockSpec((NB, D_IN), lambda i: (i, 0)),
            pl.BlockSpec((D_IN, D), lambda i: (0, 0)),
            pl.BlockSpec((1, D), lambda i: (0, 0)),
            pl.BlockSpec((D, D), lambda i: (0, 0)),
        ],
        out_specs=[
            pl.BlockSpec((NB, D), lambda i: (i, 0)),
            pl.BlockSpec((NB, D), lambda i: (i, 0)),
        ],
        out_shape=[
            jax.ShapeDtypeStruct((N, D), jnp.float32),
            jax.ShapeDtypeStruct((N, D), jnp.float32),
        ],
    )(x, We, be2, Wma)

    EBH = 2000
    nhalf = (E // 2) // EBH
    ec = pl.pallas_call(
        _pre_edges_body,
        grid=(nhalf,),
        in_specs=[
            pl.BlockSpec((EBH, D_EDGE_IN), lambda i: (i, 0)),
            pl.BlockSpec((EBH, D_EDGE_IN), lambda i, n=nhalf: (i + n, 0)),
            pl.BlockSpec((D_EDGE_IN, D), lambda i: (0, 0)),
            pl.BlockSpec((1, D), lambda i: (0, 0)),
            pl.BlockSpec((D, D), lambda i: (0, 0)),
            pl.BlockSpec((1, D), lambda i: (0, 0)),
        ],
        out_specs=pl.BlockSpec((EBH, 128), lambda i: (i, 0)),
        out_shape=jax.ShapeDtypeStruct((E // 2, 128), jnp.float32),
    )(edge_attr, edge_attr, Wel, bel2, Wmb, bm2)
    ec = ec.reshape(NW, NCHUNK, C, D)

    update = pl.pallas_call(
        _update_body,
        out_shape=[
            jax.ShapeDtypeStruct((N, D), jnp.float32),
            jax.ShapeDtypeStruct((N, D), jnp.float32),
        ],
    )

    for _ in range(3):
        mp = _sc_step(h, ec, src, dst)
        out, h = update(out, mp, Wut, Wub, bu2, Wma)
    return out


# speedup vs baseline: 1.1077x; 1.1077x over previous
import functools

import jax
import jax.numpy as jnp
from jax import lax
from jax.experimental import pallas as pl
from jax.experimental.pallas import tpu as pltpu
from jax.experimental.pallas import tpu_sc as plsc

N = 10000
E = 320000
D = 64
D_IN = 128
D_EDGE_IN = 16

NC = 2
NS = 16
NW = NC * NS
EPW = E // NW
C = 80
NCHUNK = EPW // C
RPT = 632
N_PAD = NS * RPT
LANES = 16
VPR = D // LANES


def _pre_nodes_body(x_ref, we_ref, be_ref, wma_ref, out_ref, h_ref):
    o = jnp.dot(x_ref[...], we_ref[...], preferred_element_type=jnp.float32)
    o = jnp.maximum(o + be_ref[...], 0.0)
    out_ref[...] = o
    h_ref[...] = jnp.dot(o, wma_ref[...], preferred_element_type=jnp.float32)


def _pre_edges_body(ea_ref, wel_ref, bel_ref, wmb_ref, bm_ref, ec_ref):
    def ec_half(ea):
        e = jnp.dot(ea, wel_ref[...], preferred_element_type=jnp.float32)
        e = jnp.maximum(e + bel_ref[...], 0.0)
        return jnp.dot(e, wmb_ref[...],
                       preferred_element_type=jnp.float32) + bm_ref[...]
    ec_ref[...] = jnp.concatenate(
        [ec_half(ea_ref[0]), ec_half(ea_ref[1])], axis=1)


def _update_body(out_ref, mp_ref, wut_ref, wub_ref, bu_ref, wma_ref,
                 outn_ref, hn_ref):
    m = mp_ref[0, :N] + mp_ref[1, :N]
    o = (jnp.dot(out_ref[...], wut_ref[...], preferred_element_type=jnp.float32)
         + jnp.dot(m, wub_ref[...], preferred_element_type=jnp.float32)
         + bu_ref[...])
    outn_ref[...] = o
    hn_ref[...] = jnp.dot(o, wma_ref[...], preferred_element_type=jnp.float32)


NBUF = 5


ZROWS = RPT - (RPT // C) * C


def _sc_step_body(h_hbm, ec_hbm, src_hbm, dst_hbm, out_hbm,
                  srcv, dstv, rows, macc, sem_g, sem_e, sem_s,
                  sem_i, sem_z):
    cid = lax.axis_index("c")
    sid = lax.axis_index("s")
    wid = sid * NC + cid

    cp_src = pltpu.async_copy(src_hbm.at[wid], srcv, sem_i)
    cp_dst = pltpu.async_copy(dst_hbm.at[wid], dstv, sem_i)

    def _zero(i, carry):
        rows[3, i // VPR, pl.ds((i % VPR) * LANES, LANES)] = jnp.zeros(
            (LANES,), jnp.float32)
        return carry
    lax.fori_loop(0, C * VPR, _zero, 0)

    def _zdma(z):
        if z < RPT // C:
            return pltpu.make_async_copy(
                rows.at[3], macc.at[pl.ds(sid * RPT + z * C, C)], sem_z)
        return pltpu.make_async_copy(
            rows.at[3, pl.ds(0, ZROWS)],
            macc.at[pl.ds(sid * RPT + (RPT // C) * C, ZROWS)], sem_z)
    for z in range(RPT // C + 1):
        _zdma(z).start()
    cp_src.wait()
    cp_dst.wait()

    def _drain_rows(sem, x):
        pltpu.make_async_copy(h_hbm.at[pl.ds(0, C)], rows.at[x], sem).wait()

    def _issue_ec(j, x):
        pltpu.async_copy(ec_hbm.at[wid, j], rows.at[x], sem_e.at[x])

    def _issue_gather(j, x):
        pltpu.async_copy(h_hbm.at[srcv.at[j]], rows.at[x], sem_g.at[x],
                         add=True)

    _issue_ec(0, 0)
    _issue_ec(1, 1)
    _drain_rows(sem_e.at[0], 0)
    _issue_gather(0, 0)
    for z in range(RPT // C + 1):
        _zdma(z).wait()
    plsc.subcore_barrier()

    def _outer(i, carry):
        j0 = i * NBUF
        for b in range(NBUF):
            j = j0 + b
            x2 = (b + 2) % NBUF
            x1 = (b + 1) % NBUF

            @pl.when(j + 2 < NCHUNK)
            def _():
                @pl.when(j >= 3)
                def _():
                    _drain_rows(sem_s.at[x2], x2)
                _issue_ec(j + 2, x2)

            @pl.when(j + 1 < NCHUNK)
            def _():
                _drain_rows(sem_e.at[x1], x1)
                _issue_gather(j + 1, x1)

            _drain_rows(sem_g.at[b], b)

            def _row(r, c2):
                for col in (0, 16, 32, 48):
                    rows[b, r, pl.ds(col, LANES)] = jnp.maximum(
                        rows[b, r, pl.ds(col, LANES)], 0.0)
                return c2
            lax.fori_loop(0, C, _row, 0, unroll=8)

            pltpu.async_copy(rows.at[b], macc.at[dstv.at[j]], sem_s.at[b],
                             add=True)
        return carry
    lax.fori_loop(0, NCHUNK // NBUF, _outer, 0)

    for b in range(NBUF):
        _drain_rows(sem_s.at[b], b)
    plsc.subcore_barrier()
    pltpu.sync_copy(macc.at[pl.ds(sid * RPT, RPT)],
                    out_hbm.at[cid, pl.ds(sid * RPT, RPT)])


_sc_step = functools.partial(
    pl.kernel,
    out_type=jax.ShapeDtypeStruct((NC, N_PAD, D), jnp.float32),
    mesh=plsc.VectorSubcoreMesh(core_axis_name="c", subcore_axis_name="s"),
    compiler_params=pltpu.CompilerParams(use_tc_tiling_on_sc=False),
    scratch_types=[
        pltpu.VMEM((NCHUNK, C), jnp.int32),
        pltpu.VMEM((NCHUNK, C), jnp.int32),
        pltpu.VMEM((NBUF, C, D), jnp.float32),
        pltpu.VMEM_SHARED((N_PAD, D), jnp.float32),
        pltpu.SemaphoreType.DMA((NBUF,)),
        pltpu.SemaphoreType.DMA((NBUF,)),
        pltpu.SemaphoreType.DMA((NBUF,)),
        pltpu.SemaphoreType.DMA,
        pltpu.SemaphoreType.DMA,
    ],
)(_sc_step_body)


def kernel(x, edge_index, edge_attr, We, be, Wel, bel, Wm, bm, Wu, bu):
    def perm(ix):
        ix = ix.astype(jnp.int32)
        ix = jnp.stack([ix[:E // 2], ix[E // 2:]], axis=1)
        return ix.reshape(NW, NCHUNK, C)
    src = perm(edge_index[0])
    dst = perm(edge_index[1])
    Wma, Wmb = Wm[:D], Wm[D:]
    Wut, Wub = Wu[:D], Wu[D:]
    be2 = be.reshape(1, D)
    bel2 = bel.reshape(1, D)
    bm2 = bm.reshape(1, D)
    bu2 = bu.reshape(1, D)

    NB = 1000
    out, h = pl.pallas_call(
        _pre_nodes_body,
        grid=(N // NB,),
        in_specs=[
            pl.BlockSpec((NB, D_IN), lambda i: (i, 0)),
            pl.BlockSpec((D_IN, D), lambda i: (0, 0)),
            pl.BlockSpec((1, D), lambda i: (0, 0)),
            pl.BlockSpec((D, D), lambda i: (0, 0)),
        ],
        out_specs=[
            pl.BlockSpec((NB, D), lambda i: (i, 0)),
            pl.BlockSpec((NB, D), lambda i: (i, 0)),
        ],
        out_shape=[
            jax.ShapeDtypeStruct((N, D), jnp.float32),
            jax.ShapeDtypeStruct((N, D), jnp.float32),
        ],
    )(x, We, be2, Wma)

    EBH = 2000
    nhalf = (E // 2) // EBH
    ec = pl.pallas_call(
        _pre_edges_body,
        grid=(nhalf,),
        in_specs=[
            pl.BlockSpec((2, EBH, D_EDGE_IN), lambda i: (0, i, 0)),
            pl.BlockSpec((D_EDGE_IN, D), lambda i: (0, 0)),
            pl.BlockSpec((1, D), lambda i: (0, 0)),
            pl.BlockSpec((D, D), lambda i: (0, 0)),
            pl.BlockSpec((1, D), lambda i: (0, 0)),
        ],
        out_specs=pl.BlockSpec((EBH, 2 * D), lambda i: (i, 0)),
        out_shape=jax.ShapeDtypeStruct((E // 2, 2 * D), jnp.float32),
    )(edge_attr.reshape(2, E // 2, D_EDGE_IN), Wel, bel2, Wmb, bm2)
    ec = ec.reshape(NW, NCHUNK, C, D)

    update = pl.pallas_call(
        _update_body,
        out_shape=[
            jax.ShapeDtypeStruct((N, D), jnp.float32),
            jax.ShapeDtypeStruct((N, D), jnp.float32),
        ],
    )

    for _ in range(3):
        mp = _sc_step(h, ec, src, dst)
        out, h = update(out, mp, Wut, Wub, bu2, Wma)
    return out


# speedup vs baseline: 1.2764x; 1.1523x over previous
import functools

import jax
import jax.numpy as jnp
from jax import lax
from jax.experimental import pallas as pl
from jax.experimental.pallas import tpu as pltpu
from jax.experimental.pallas import tpu_sc as plsc

N = 10000
E = 320000
D = 64
D_IN = 128
D_EDGE_IN = 16

NC = 2
NS = 16
NW = NC * NS
EPW = E // NW
C = 80
NCHUNK = EPW // C
RPT = 632
N_PAD = NS * RPT
LANES = 16
VPR = D // LANES


def _pre_nodes_body(x_ref, we_ref, be_ref, wma_ref, out_ref, h_ref):
    o = jnp.dot(x_ref[...], we_ref[...], preferred_element_type=jnp.float32)
    o = jnp.maximum(o + be_ref[...], 0.0)
    out_ref[...] = o
    h_ref[...] = jnp.dot(o, wma_ref[...], preferred_element_type=jnp.float32)


def _pre_edges_body(ea_ref, wel_ref, bel_ref, wmb_ref, bm_ref, ec_ref):
    def ec_half(ea):
        e = jnp.dot(ea, wel_ref[...], preferred_element_type=jnp.float32)
        e = jnp.maximum(e + bel_ref[...], 0.0)
        return jnp.dot(e, wmb_ref[...],
                       preferred_element_type=jnp.float32) + bm_ref[...]
    ec_ref[...] = jnp.concatenate(
        [ec_half(ea_ref[:, :D_EDGE_IN]), ec_half(ea_ref[:, D_EDGE_IN:])],
        axis=1)


def _update_body(out_ref, mp_ref, wut_ref, wub_ref, bu_ref, wma_ref,
                 outn_ref, hn_ref):
    m = mp_ref[0, :N] + mp_ref[1, :N]
    o = (jnp.dot(out_ref[...], wut_ref[...], preferred_element_type=jnp.float32)
         + jnp.dot(m, wub_ref[...], preferred_element_type=jnp.float32)
         + bu_ref[...])
    outn_ref[...] = o
    hn_ref[...] = jnp.dot(o, wma_ref[...], preferred_element_type=jnp.float32)


NBUF = 5


ZROWS = RPT - (RPT // C) * C


def _sc_step_body(h_hbm, ec_hbm, src_hbm, dst_hbm, out_hbm,
                  srcv, dstv, rows, h_sh, macc, sem_g, sem_e, sem_s,
                  sem_i, sem_z):
    cid = lax.axis_index("c")
    sid = lax.axis_index("s")
    wid = sid * NC + cid

    @pl.when(sid < NS - 1)
    def _():
        cp = pltpu.make_async_copy(h_hbm.at[pl.ds(sid * RPT, RPT)],
                                   h_sh.at[pl.ds(sid * RPT, RPT)], sem_z)
        cp.start()

    @pl.when(sid == NS - 1)
    def _():
        cp = pltpu.make_async_copy(
            h_hbm.at[pl.ds((NS - 1) * RPT, N - (NS - 1) * RPT)],
            h_sh.at[pl.ds((NS - 1) * RPT, N - (NS - 1) * RPT)], sem_z)
        cp.start()

    cp_src = pltpu.async_copy(src_hbm.at[wid], srcv, sem_i)
    cp_dst = pltpu.async_copy(dst_hbm.at[wid], dstv, sem_i)

    def _zero(i, carry):
        rows[3, i // VPR, pl.ds((i % VPR) * LANES, LANES)] = jnp.zeros(
            (LANES,), jnp.float32)
        return carry
    lax.fori_loop(0, C * VPR, _zero, 0)

    def _zdma(z):
        if z < RPT // C:
            return pltpu.make_async_copy(
                rows.at[3], macc.at[pl.ds(sid * RPT + z * C, C)], sem_z)
        return pltpu.make_async_copy(
            rows.at[3, pl.ds(0, ZROWS)],
            macc.at[pl.ds(sid * RPT + (RPT // C) * C, ZROWS)], sem_z)
    for z in range(RPT // C + 1):
        _zdma(z).start()
    cp_src.wait()
    cp_dst.wait()

    def _drain_rows(sem, x):
        pltpu.make_async_copy(h_hbm.at[pl.ds(0, C)], rows.at[x], sem).wait()

    def _issue_ec(j, x):
        pltpu.async_copy(ec_hbm.at[wid, j], rows.at[x], sem_e.at[x])

    def _issue_gather(j, x):
        pltpu.async_copy(h_sh.at[srcv.at[j]], rows.at[x], sem_g.at[x],
                         add=True)

    _issue_ec(0, 0)
    _issue_ec(1, 1)
    for z in range(RPT // C + 1):
        _zdma(z).wait()

    @pl.when(sid < NS - 1)
    def _():
        pltpu.make_async_copy(h_hbm.at[pl.ds(sid * RPT, RPT)],
                              h_sh.at[pl.ds(sid * RPT, RPT)], sem_z).wait()

    @pl.when(sid == NS - 1)
    def _():
        pltpu.make_async_copy(
            h_hbm.at[pl.ds((NS - 1) * RPT, N - (NS - 1) * RPT)],
            h_sh.at[pl.ds((NS - 1) * RPT, N - (NS - 1) * RPT)], sem_z).wait()

    plsc.subcore_barrier()
    _drain_rows(sem_e.at[0], 0)
    _issue_gather(0, 0)

    def _outer(i, carry):
        j0 = i * NBUF
        for b in range(NBUF):
            j = j0 + b
            x2 = (b + 2) % NBUF
            x1 = (b + 1) % NBUF

            @pl.when(j + 2 < NCHUNK)
            def _():
                @pl.when(j >= 3)
                def _():
                    _drain_rows(sem_s.at[x2], x2)
                _issue_ec(j + 2, x2)

            @pl.when(j + 1 < NCHUNK)
            def _():
                _drain_rows(sem_e.at[x1], x1)
                _issue_gather(j + 1, x1)

            _drain_rows(sem_g.at[b], b)

            def _row(r, c2):
                for col in (0, 16, 32, 48):
                    rows[b, r, pl.ds(col, LANES)] = jnp.maximum(
                        rows[b, r, pl.ds(col, LANES)], 0.0)
                return c2
            lax.fori_loop(0, C, _row, 0, unroll=8)

            pltpu.async_copy(rows.at[b], macc.at[dstv.at[j]], sem_s.at[b],
                             add=True)
        return carry
    lax.fori_loop(0, NCHUNK // NBUF, _outer, 0)

    for b in range(NBUF):
        _drain_rows(sem_s.at[b], b)
    plsc.subcore_barrier()
    pltpu.sync_copy(macc.at[pl.ds(sid * RPT, RPT)],
                    out_hbm.at[cid, pl.ds(sid * RPT, RPT)])


_sc_step = functools.partial(
    pl.kernel,
    out_type=jax.ShapeDtypeStruct((NC, N_PAD, D), jnp.float32),
    mesh=plsc.VectorSubcoreMesh(core_axis_name="c", subcore_axis_name="s"),
    compiler_params=pltpu.CompilerParams(use_tc_tiling_on_sc=False),
    scratch_types=[
        pltpu.VMEM((NCHUNK, C), jnp.int32),
        pltpu.VMEM((NCHUNK, C), jnp.int32),
        pltpu.VMEM((NBUF, C, D), jnp.float32),
        pltpu.VMEM_SHARED((N, D), jnp.float32),
        pltpu.VMEM_SHARED((N_PAD, D), jnp.float32),
        pltpu.SemaphoreType.DMA((NBUF,)),
        pltpu.SemaphoreType.DMA((NBUF,)),
        pltpu.SemaphoreType.DMA((NBUF,)),
        pltpu.SemaphoreType.DMA,
        pltpu.SemaphoreType.DMA,
    ],
)(_sc_step_body)


def kernel(x, edge_index, edge_attr, We, be, Wel, bel, Wm, bm, Wu, bu):
    src = edge_index[0].astype(jnp.int32).reshape(NW, NCHUNK, C)
    dst = edge_index[1].astype(jnp.int32).reshape(NW, NCHUNK, C)
    Wma, Wmb = Wm[:D], Wm[D:]
    Wut, Wub = Wu[:D], Wu[D:]
    be2 = be.reshape(1, D)
    bel2 = bel.reshape(1, D)
    bm2 = bm.reshape(1, D)
    bu2 = bu.reshape(1, D)

    NB = 1000
    out, h = pl.pallas_call(
        _pre_nodes_body,
        grid=(N // NB,),
        in_specs=[
            pl.BlockSpec((NB, D_IN), lambda i: (i, 0)),
            pl.BlockSpec((D_IN, D), lambda i: (0, 0)),
            pl.BlockSpec((1, D), lambda i: (0, 0)),
            pl.BlockSpec((D, D), lambda i: (0, 0)),
        ],
        out_specs=[
            pl.BlockSpec((NB, D), lambda i: (i, 0)),
            pl.BlockSpec((NB, D), lambda i: (i, 0)),
        ],
        out_shape=[
            jax.ShapeDtypeStruct((N, D), jnp.float32),
            jax.ShapeDtypeStruct((N, D), jnp.float32),
        ],
    )(x, We, be2, Wma)

    EB = 4000
    ec = pl.pallas_call(
        _pre_edges_body,
        grid=(E // EB,),
        in_specs=[
            pl.BlockSpec((EB // 2, 2 * D_EDGE_IN), lambda i: (i, 0)),
            pl.BlockSpec((D_EDGE_IN, D), lambda i: (0, 0)),
            pl.BlockSpec((1, D), lambda i: (0, 0)),
            pl.BlockSpec((D, D), lambda i: (0, 0)),
            pl.BlockSpec((1, D), lambda i: (0, 0)),
        ],
        out_specs=pl.BlockSpec((EB // 2, 128), lambda i: (i, 0)),
        out_shape=jax.ShapeDtypeStruct((E // 2, 128), jnp.float32),
    )(edge_attr.reshape(E // 2, 2 * D_EDGE_IN), Wel, bel2, Wmb, bm2)
    ec = ec.reshape(NW, NCHUNK, C, D)

    update = pl.pallas_call(
        _update_body,
        out_shape=[
            jax.ShapeDtypeStruct((N, D), jnp.float32),
            jax.ShapeDtypeStruct((N, D), jnp.float32),
        ],
    )

    for _ in range(3):
        mp = _sc_step(h, ec, src, dst)
        out, h = update(out, mp, Wut, Wub, bu2, Wma)
    return out
